# baseline (speedup 1.0000x reference)
import dataclasses
import functools

import numpy as np

import jax
import jax.numpy as jnp
from jax import lax
from jax.experimental import pallas as pl
from jax.experimental.pallas import tpu as pltpu
from jax.experimental.pallas import tpu_sc as plsc

N = 10000
NP = 10240
E = 320000
T = 5000
TP = 5120
D = 128
NC, NS = 2, 16
NW = NC * NS
EPW = E // NW
K = 80
CH = 128
SB = 64
NSB = CH // SB
ROWS_PER_SUB = NP // NS
BR = 1024

_HI = jax.lax.Precision.HIGHEST

def _gumbel_const():
    g = jax.random.gumbel(jax.random.key(42), (T,), jnp.float32)
    return np.pad(np.asarray(g), (0, TP - T)).reshape(TP // D, D)

_GUM2D = _gumbel_const()


def _tc0_body(nt_ref, wemb_ref, bemb_ref, wt_ref, h_ref, g_ref):
    t = nt_ref[...]
    w0 = wemb_ref[0:1, :]
    w1 = wemb_ref[1:2, :]
    w2 = wemb_ref[2:3, :]
    h = jnp.where(t == 0, w0, jnp.where(t == 1, w1, w2)) + bemb_ref[...]
    h_ref[...] = h
    g_ref[...] = lax.dot_general(h, wt_ref[...], (((1,), (0,)), ((), ())),
                                 precision=_HI, preferred_element_type=jnp.float32)


def _tc_embed_matmul(nt2, wembT, bemb, wT):
    return pl.pallas_call(
        _tc0_body,
        grid=(NP // BR,),
        in_specs=[
            pl.BlockSpec((BR, 1), lambda i: (i, 0)),
            pl.BlockSpec((3, D), lambda i: (0, 0)),
            pl.BlockSpec((1, D), lambda i: (0, 0)),
            pl.BlockSpec((D, D), lambda i: (0, 0)),
        ],
        out_specs=[
            pl.BlockSpec((BR, D), lambda i: (i, 0)),
            pl.BlockSpec((BR, D), lambda i: (i, 0)),
        ],
        out_shape=[
            jax.ShapeDtypeStruct((NP, D), jnp.float32),
            jax.ShapeDtypeStruct((NP, D), jnp.float32),
        ],
    )(nt2, wembT, bemb, wT)


def _tc_mid_body(h_ref, p0_ref, p1_ref, b_ref, wt_ref, hn_ref, g_ref):
    h = h_ref[...] + jnp.maximum(p0_ref[...] + p1_ref[...] + b_ref[...], 0.0)
    hn_ref[...] = h
    g_ref[...] = lax.dot_general(h, wt_ref[...], (((1,), (0,)), ((), ())),
                                 precision=_HI, preferred_element_type=jnp.float32)


def _tc_update_matmul(h, p0, p1, b, wT):
    return pl.pallas_call(
        _tc_mid_body,
        grid=(NP // BR,),
        in_specs=[
            pl.BlockSpec((BR, D), lambda i: (i, 0)),
            pl.BlockSpec((BR, D), lambda i: (i, 0)),
            pl.BlockSpec((BR, D), lambda i: (i, 0)),
            pl.BlockSpec((1, D), lambda i: (0, 0)),
            pl.BlockSpec((D, D), lambda i: (0, 0)),
        ],
        out_specs=[
            pl.BlockSpec((BR, D), lambda i: (i, 0)),
            pl.BlockSpec((BR, D), lambda i: (i, 0)),
        ],
        out_shape=[
            jax.ShapeDtypeStruct((NP, D), jnp.float32),
            jax.ShapeDtypeStruct((NP, D), jnp.float32),
        ],
    )(h, p0, p1, b, wT)


def _tc3_body(ag_ref, h_ref, p0_ref, p1_ref, b_ref, s_ref):
    ag = ag_ref[0, 0]
    b = b_ref[...]
    h3 = h_ref[...] + jnp.maximum(p0_ref[...] + p1_ref[...] + b, 0.0)
    agrow = h_ref[pl.ds(ag, 1), :] + jnp.maximum(
        p0_ref[pl.ds(ag, 1), :] + p1_ref[pl.ds(ag, 1), :] + b, 0.0)
    s_ref[...] = lax.dot_general(h3, agrow, (((1,), (1,)), ((), ())),
                                 precision=_HI, preferred_element_type=jnp.float32)


def _tc_final_update_score(ag2, h, p0, p1, b):
    return pl.pallas_call(
        _tc3_body,
        in_specs=[
            pl.BlockSpec(memory_space=pltpu.SMEM),
            pl.BlockSpec((NP, D), lambda: (0, 0)),
            pl.BlockSpec((NP, D), lambda: (0, 0)),
            pl.BlockSpec((NP, D), lambda: (0, 0)),
            pl.BlockSpec((1, D), lambda: (0, 0)),
        ],
        out_specs=pl.BlockSpec((NP, 1), lambda: (0, 0)),
        out_shape=jax.ShapeDtypeStruct((NP, 1), jnp.float32),
    )(ag2, h, p0, p1, b)


def _tc4_body(sc_ref, mask_ref, gum_ref, pi_ref, act_ref):
    inv = jnp.float32(1.0) / jnp.sqrt(jnp.float32(D))
    score = jnp.where(mask_ref[...] != 0, -jnp.inf, sc_ref[...] * inv)
    m = jnp.max(score)
    e = jnp.exp(score - m)
    pi_ref[...] = e / jnp.sum(e)
    v = score + gum_ref[...]
    m2 = jnp.max(v)
    r = lax.broadcasted_iota(jnp.int32, (TP // D, D), 0)
    c = lax.broadcasted_iota(jnp.int32, (TP // D, D), 1)
    flat = r * D + c
    act_ref[0, 0] = jnp.min(jnp.where(v == m2, flat, jnp.int32(2**30)))


def _tc_softmax_sample(sc2, mask2, gum2):
    return pl.pallas_call(
        _tc4_body,
        in_specs=[
            pl.BlockSpec((TP // D, D), lambda: (0, 0)),
            pl.BlockSpec((TP // D, D), lambda: (0, 0)),
            pl.BlockSpec((TP // D, D), lambda: (0, 0)),
        ],
        out_specs=[
            pl.BlockSpec((TP // D, D), lambda: (0, 0)),
            pl.BlockSpec(memory_space=pltpu.SMEM),
        ],
        out_shape=[
            jax.ShapeDtypeStruct((TP // D, D), jnp.float32),
            jax.ShapeDtypeStruct((1, 1), jnp.int32),
        ],
    )(sc2, mask2, gum2)


_MESH = plsc.VectorSubcoreMesh(core_axis_name="c", subcore_axis_name="s")


@functools.partial(
    pl.kernel,
    out_type=jax.ShapeDtypeStruct((NC, NP, D), jnp.float32),
    mesh=_MESH,
    scratch_types=[
        pltpu.VMEM((SB, K), jnp.int32),
        pltpu.VMEM((SB, K), jnp.int32),
        pltpu.VMEM((K, D), jnp.float32),
        pltpu.VMEM((K, D), jnp.float32),
        pltpu.VMEM_SHARED((NP, D), jnp.float32),
    ],
)
def _sc_segsum(g_hbm, src_hbm, dst_hbm, out_hbm, sidx, didx, rows, rows1, acc):
    cid = lax.axis_index("c")
    sid = lax.axis_index("s")
    wid = cid * NS + sid

    @pl.loop(0, K)
    def _(i):
        @pl.loop(0, D // 16)
        def _(j):
            rows.at[i][pl.ds(j * 16, 16)] = jnp.zeros((16,), jnp.float32)

    @pl.loop(0, ROWS_PER_SUB // K)
    def _(i):
        pltpu.sync_copy(rows, acc.at[pl.ds(sid * ROWS_PER_SUB + i * K, K)])

    plsc.subcore_barrier()

    @pl.loop(0, NSB)
    def _(sb):
        pltpu.sync_copy(src_hbm.at[wid].at[pl.ds(sb * SB, SB)], sidx)
        pltpu.sync_copy(dst_hbm.at[wid].at[pl.ds(sb * SB, SB)], didx)

        @pl.loop(0, SB, step=2)
        def _(c):
            def inner(g0, g1, s0, s1):
                d0 = pltpu.make_async_copy(g_hbm.at[sidx.at[c]], rows, g0)
                d1 = pltpu.make_async_copy(g_hbm.at[sidx.at[c + 1]], rows1, g1)
                d0.start()
                d1.start()
                d0.wait()
                w0 = pltpu.make_async_copy(rows, acc.at[didx.at[c]], s0)
                w0.start(add=True)
                d1.wait()
                w1 = pltpu.make_async_copy(rows1, acc.at[didx.at[c + 1]], s1)
                w1.start(add=True)
                w0.wait()
                w1.wait()
            pl.run_scoped(inner,
                          pltpu.SemaphoreType.DMA(()), pltpu.SemaphoreType.DMA(()),
                          pltpu.SemaphoreType.DMA(()), pltpu.SemaphoreType.DMA(()))

    plsc.subcore_barrier()

    pltpu.sync_copy(acc.at[pl.ds(sid * ROWS_PER_SUB, ROWS_PER_SUB)],
                    out_hbm.at[cid].at[pl.ds(sid * ROWS_PER_SUB, ROWS_PER_SUB)])


TIPW = TP // NW

_CP = pltpu.CompilerParams()
if "needs_layout_passes" in pltpu.CompilerParams.__dataclass_fields__:
    _CP = dataclasses.replace(_CP, needs_layout_passes=False)


@functools.partial(
    pl.kernel,
    out_type=jax.ShapeDtypeStruct((TP,), jnp.float32),
    mesh=_MESH,
    compiler_params=_CP,
    scratch_types=[
        pltpu.VMEM((NP,), jnp.float32),
        pltpu.VMEM((TIPW,), jnp.int32),
        pltpu.VMEM((TIPW,), jnp.float32),
    ],
)
def _sc_score_gather(s_hbm, ti_hbm, out_hbm, sv, tiv, ov):
    cid = lax.axis_index("c")
    sid = lax.axis_index("s")
    wid = cid * NS + sid
    pltpu.sync_copy(s_hbm, sv)
    pltpu.sync_copy(ti_hbm.at[wid], tiv)

    @pl.loop(0, TIPW // 16)
    def _(j):
        iv = tiv[pl.ds(j * 16, 16)]
        ov[pl.ds(j * 16, 16)] = plsc.load_gather(sv, [iv])

    pltpu.sync_copy(ov, out_hbm.at[pl.ds(wid * TIPW, TIPW)])


def kernel(node_type, edge_index, ag_node_idx, task_node_indices, finished_task,
           W_emb, b_emb, W_gnn, b_gnn):
    f32, i32 = jnp.float32, jnp.int32
    nt2 = jnp.pad(node_type.astype(i32), (0, NP - N)).reshape(NP, 1)
    pad_e = CH * K - EPW
    src3 = jnp.pad(edge_index[0].astype(i32).reshape(NW, EPW),
                   ((0, 0), (0, pad_e)), constant_values=NP - 1).reshape(NW, CH, K)
    dst3 = jnp.pad(edge_index[1].astype(i32).reshape(NW, EPW),
                   ((0, 0), (0, pad_e)), constant_values=NP - 1).reshape(NW, CH, K)
    wembT = W_emb.T.astype(f32)
    bemb = b_emb.reshape(1, D).astype(f32)

    h, g = _tc_embed_matmul(nt2, wembT, bemb, W_gnn[0].T)
    for l in range(W_gnn.shape[0]):
        p = _sc_segsum(g, src3, dst3)
        b_l = b_gnn[l].reshape(1, D)
        if l + 1 < W_gnn.shape[0]:
            h, g = _tc_update_matmul(h, p[0], p[1], b_l, W_gnn[l + 1].T)
        else:
            ag2 = jnp.asarray(ag_node_idx, i32).reshape(1, 1)
            s = _tc_final_update_score(ag2, h, p[0], p[1], b_l)

    ti2 = jnp.pad(task_node_indices.astype(i32), (0, TP - T)).reshape(NW, TIPW)
    sc = _sc_score_gather(s.reshape(NP), ti2)

    mask2 = jnp.pad(finished_task, (0, TP - T), constant_values=True)
    mask2 = mask2.astype(i32).reshape(TP // D, D)
    gum2 = jnp.asarray(_GUM2D)
    pi2, act = _tc_softmax_sample(sc.reshape(TP // D, D), mask2, gum2)

    pi = pi2.reshape(TP)[:T, None]
    return act[0, 0], pi


# speedup vs baseline: 1.9759x; 1.9759x over previous
import dataclasses
import functools

import numpy as np

import jax
import jax.numpy as jnp
from jax import lax
from jax.experimental import pallas as pl
from jax.experimental.pallas import tpu as pltpu
from jax.experimental.pallas import tpu_sc as plsc

N = 10000
NP = 10240
E = 320000
T = 5000
TP = 5120
D = 128
NC, NS = 2, 16
NW = NC * NS
EPW = E // NW
K = 80
CH = 125
ROWS_PER_SUB = NP // NS
BR = 1024

_HI = jax.lax.Precision.HIGHEST

def _gumbel_const():
    g = jax.random.gumbel(jax.random.key(42), (T,), jnp.float32)
    return np.pad(np.asarray(g), (0, TP - T)).reshape(TP // D, D)

_GUM2D = _gumbel_const()


def _tc0_body(nt_ref, wemb_ref, bemb_ref, wt_ref, h_ref, g_ref):
    t = nt_ref[...]
    w0 = wemb_ref[0:1, :]
    w1 = wemb_ref[1:2, :]
    w2 = wemb_ref[2:3, :]
    h = jnp.where(t == 0, w0, jnp.where(t == 1, w1, w2)) + bemb_ref[...]
    h_ref[...] = h
    g_ref[...] = lax.dot_general(h, wt_ref[...], (((1,), (0,)), ((), ())),
                                 precision=_HI, preferred_element_type=jnp.float32)


def _tc_embed_matmul(nt2, wembT, bemb, wT):
    return pl.pallas_call(
        _tc0_body,
        grid=(NP // BR,),
        in_specs=[
            pl.BlockSpec((BR, 1), lambda i: (i, 0)),
            pl.BlockSpec((3, D), lambda i: (0, 0)),
            pl.BlockSpec((1, D), lambda i: (0, 0)),
            pl.BlockSpec((D, D), lambda i: (0, 0)),
        ],
        out_specs=[
            pl.BlockSpec((BR, D), lambda i: (i, 0)),
            pl.BlockSpec((BR, D), lambda i: (i, 0)),
        ],
        out_shape=[
            jax.ShapeDtypeStruct((NP, D), jnp.float32),
            jax.ShapeDtypeStruct((NP, D), jnp.float32),
        ],
    )(nt2, wembT, bemb, wT)


def _tc_mid_body(h_ref, p0_ref, p1_ref, b_ref, wt_ref, hn_ref, g_ref):
    h = h_ref[...] + jnp.maximum(p0_ref[...] + p1_ref[...] + b_ref[...], 0.0)
    hn_ref[...] = h
    g_ref[...] = lax.dot_general(h, wt_ref[...], (((1,), (0,)), ((), ())),
                                 precision=_HI, preferred_element_type=jnp.float32)


def _tc_update_matmul(h, p0, p1, b, wT):
    return pl.pallas_call(
        _tc_mid_body,
        grid=(NP // BR,),
        in_specs=[
            pl.BlockSpec((BR, D), lambda i: (i, 0)),
            pl.BlockSpec((BR, D), lambda i: (i, 0)),
            pl.BlockSpec((BR, D), lambda i: (i, 0)),
            pl.BlockSpec((1, D), lambda i: (0, 0)),
            pl.BlockSpec((D, D), lambda i: (0, 0)),
        ],
        out_specs=[
            pl.BlockSpec((BR, D), lambda i: (i, 0)),
            pl.BlockSpec((BR, D), lambda i: (i, 0)),
        ],
        out_shape=[
            jax.ShapeDtypeStruct((NP, D), jnp.float32),
            jax.ShapeDtypeStruct((NP, D), jnp.float32),
        ],
    )(h, p0, p1, b, wT)


def _tc3_body(ag_ref, h_ref, p0_ref, p1_ref, b_ref, s_ref):
    ag = ag_ref[0, 0]
    b = b_ref[...]
    h3 = h_ref[...] + jnp.maximum(p0_ref[...] + p1_ref[...] + b, 0.0)
    agrow = h_ref[pl.ds(ag, 1), :] + jnp.maximum(
        p0_ref[pl.ds(ag, 1), :] + p1_ref[pl.ds(ag, 1), :] + b, 0.0)
    s_ref[...] = lax.dot_general(h3, agrow, (((1,), (1,)), ((), ())),
                                 precision=_HI, preferred_element_type=jnp.float32)


def _tc_final_update_score(ag2, h, p0, p1, b):
    return pl.pallas_call(
        _tc3_body,
        in_specs=[
            pl.BlockSpec(memory_space=pltpu.SMEM),
            pl.BlockSpec((NP, D), lambda: (0, 0)),
            pl.BlockSpec((NP, D), lambda: (0, 0)),
            pl.BlockSpec((NP, D), lambda: (0, 0)),
            pl.BlockSpec((1, D), lambda: (0, 0)),
        ],
        out_specs=pl.BlockSpec((NP, 1), lambda: (0, 0)),
        out_shape=jax.ShapeDtypeStruct((NP, 1), jnp.float32),
    )(ag2, h, p0, p1, b)


def _tc4_body(sc_ref, mask_ref, gum_ref, pi_ref, act_ref):
    inv = jnp.float32(1.0) / jnp.sqrt(jnp.float32(D))
    score = jnp.where(mask_ref[...] != 0, -jnp.inf, sc_ref[...] * inv)
    m = jnp.max(score)
    e = jnp.exp(score - m)
    pi_ref[...] = e / jnp.sum(e)
    v = score + gum_ref[...]
    m2 = jnp.max(v)
    r = lax.broadcasted_iota(jnp.int32, (TP // D, D), 0)
    c = lax.broadcasted_iota(jnp.int32, (TP // D, D), 1)
    flat = r * D + c
    act_ref[0, 0] = jnp.min(jnp.where(v == m2, flat, jnp.int32(2**30)))


def _tc_softmax_sample(sc2, mask2, gum2):
    return pl.pallas_call(
        _tc4_body,
        in_specs=[
            pl.BlockSpec((TP // D, D), lambda: (0, 0)),
            pl.BlockSpec((TP // D, D), lambda: (0, 0)),
            pl.BlockSpec((TP // D, D), lambda: (0, 0)),
        ],
        out_specs=[
            pl.BlockSpec((TP // D, D), lambda: (0, 0)),
            pl.BlockSpec(memory_space=pltpu.SMEM),
        ],
        out_shape=[
            jax.ShapeDtypeStruct((TP // D, D), jnp.float32),
            jax.ShapeDtypeStruct((1, 1), jnp.int32),
        ],
    )(sc2, mask2, gum2)


_MESH = plsc.VectorSubcoreMesh(core_axis_name="c", subcore_axis_name="s")


@functools.partial(
    pl.kernel,
    out_type=jax.ShapeDtypeStruct((NC, NP, D), jnp.float32),
    mesh=_MESH,
    scratch_types=[
        pltpu.VMEM((CH, K), jnp.int32),
        pltpu.VMEM((CH, K), jnp.int32),
        pltpu.VMEM((K, D), jnp.float32),
        pltpu.VMEM_SHARED((NP, D), jnp.float32),
    ],
)
def _sc_segsum(g_hbm, src_hbm, dst_hbm, out_hbm, sidx, didx, rows, acc):
    cid = lax.axis_index("c")
    sid = lax.axis_index("s")
    wid = cid * NS + sid

    @pl.loop(0, K)
    def _(i):
        @pl.loop(0, D // 16)
        def _(j):
            rows.at[i][pl.ds(j * 16, 16)] = jnp.zeros((16,), jnp.float32)

    @pl.loop(0, ROWS_PER_SUB // K)
    def _(i):
        pltpu.sync_copy(rows, acc.at[pl.ds(sid * ROWS_PER_SUB + i * K, K)])

    plsc.subcore_barrier()

    pltpu.sync_copy(src_hbm.at[wid], sidx)
    pltpu.sync_copy(dst_hbm.at[wid], didx)

    @pl.loop(0, CH)
    def _(c):
        pltpu.sync_copy(g_hbm.at[sidx.at[c]], rows)
        pltpu.sync_copy(rows, acc.at[didx.at[c]], add=True)

    plsc.subcore_barrier()

    pltpu.sync_copy(acc.at[pl.ds(sid * ROWS_PER_SUB, ROWS_PER_SUB)],
                    out_hbm.at[cid].at[pl.ds(sid * ROWS_PER_SUB, ROWS_PER_SUB)])


TIPW = TP // NW

_CP = pltpu.CompilerParams()
if "needs_layout_passes" in pltpu.CompilerParams.__dataclass_fields__:
    _CP = dataclasses.replace(_CP, needs_layout_passes=False)


@functools.partial(
    pl.kernel,
    out_type=jax.ShapeDtypeStruct((TP,), jnp.float32),
    mesh=_MESH,
    compiler_params=_CP,
    scratch_types=[
        pltpu.VMEM((NP,), jnp.float32),
        pltpu.VMEM((TIPW,), jnp.int32),
        pltpu.VMEM((TIPW,), jnp.float32),
    ],
)
def _sc_score_gather(s_hbm, ti_hbm, out_hbm, sv, tiv, ov):
    cid = lax.axis_index("c")
    sid = lax.axis_index("s")
    wid = cid * NS + sid
    pltpu.sync_copy(s_hbm, sv)
    pltpu.sync_copy(ti_hbm.at[wid], tiv)

    @pl.loop(0, TIPW // 16)
    def _(j):
        iv = tiv[pl.ds(j * 16, 16)]
        ov[pl.ds(j * 16, 16)] = plsc.load_gather(sv, [iv])

    pltpu.sync_copy(ov, out_hbm.at[pl.ds(wid * TIPW, TIPW)])


def kernel(node_type, edge_index, ag_node_idx, task_node_indices, finished_task,
           W_emb, b_emb, W_gnn, b_gnn):
    f32, i32 = jnp.float32, jnp.int32
    nt2 = jnp.pad(node_type.astype(i32), (0, NP - N)).reshape(NP, 1)
    pad_e = CH * K - EPW
    src3 = jnp.pad(edge_index[0].astype(i32).reshape(NW, EPW),
                   ((0, 0), (0, pad_e)), constant_values=NP - 1).reshape(NW, CH, K)
    dst3 = jnp.pad(edge_index[1].astype(i32).reshape(NW, EPW),
                   ((0, 0), (0, pad_e)), constant_values=NP - 1).reshape(NW, CH, K)
    wembT = W_emb.T.astype(f32)
    bemb = b_emb.reshape(1, D).astype(f32)

    h, g = _tc_embed_matmul(nt2, wembT, bemb, W_gnn[0].T)
    for l in range(W_gnn.shape[0]):
        p = _sc_segsum(g, src3, dst3)
        b_l = b_gnn[l].reshape(1, D)
        if l + 1 < W_gnn.shape[0]:
            h, g = _tc_update_matmul(h, p[0], p[1], b_l, W_gnn[l + 1].T)
        else:
            ag2 = jnp.asarray(ag_node_idx, i32).reshape(1, 1)
            s = _tc_final_update_score(ag2, h, p[0], p[1], b_l)

    ti2 = jnp.pad(task_node_indices.astype(i32), (0, TP - T)).reshape(NW, TIPW)
    sc = _sc_score_gather(s.reshape(NP), ti2)

    mask2 = jnp.pad(finished_task, (0, TP - T), constant_values=True)
    mask2 = mask2.astype(i32).reshape(TP // D, D)
    gum2 = jnp.asarray(_GUM2D)
    pi2, act = _tc_softmax_sample(sc.reshape(TP // D, D), mask2, gum2)

    pi = pi2.reshape(TP)[:T, None]
    return act[0, 0], pi


# speedup vs baseline: 2.4241x; 1.2268x over previous
import dataclasses
import functools

import numpy as np

import jax
import jax.numpy as jnp
from jax import lax
from jax.experimental import pallas as pl
from jax.experimental.pallas import tpu as pltpu
from jax.experimental.pallas import tpu_sc as plsc

N = 10000
NP = 10240
E = 320000
T = 5000
TP = 5120
D = 128
NC, NS = 2, 16
NW = NC * NS
EPW = E // NW
K = 80
CH = 128
SB = 64
NSB = CH // SB
ROWS_PER_SUB = NP // NS
BR = 1024

_HI = jax.lax.Precision.HIGHEST

def _gumbel_const():
    g = jax.random.gumbel(jax.random.key(42), (T,), jnp.float32)
    return np.pad(np.asarray(g), (0, TP - T)).reshape(TP // D, D)

_GUM2D = _gumbel_const()


def _tc0_body(nt_ref, wemb_ref, bemb_ref, wt_ref, h_ref, g_ref):
    t = nt_ref[...]
    w0 = wemb_ref[0:1, :]
    w1 = wemb_ref[1:2, :]
    w2 = wemb_ref[2:3, :]
    h = jnp.where(t == 0, w0, jnp.where(t == 1, w1, w2)) + bemb_ref[...]
    h_ref[...] = h
    g_ref[...] = lax.dot_general(h, wt_ref[...], (((1,), (0,)), ((), ())),
                                 precision=_HI, preferred_element_type=jnp.float32)


def _tc_embed_matmul(nt2, wembT, bemb, wT):
    return pl.pallas_call(
        _tc0_body,
        grid=(NP // BR,),
        in_specs=[
            pl.BlockSpec((BR, 1), lambda i: (i, 0)),
            pl.BlockSpec((3, D), lambda i: (0, 0)),
            pl.BlockSpec((1, D), lambda i: (0, 0)),
            pl.BlockSpec((D, D), lambda i: (0, 0)),
        ],
        out_specs=[
            pl.BlockSpec((BR, D), lambda i: (i, 0)),
            pl.BlockSpec((BR, D), lambda i: (i, 0)),
        ],
        out_shape=[
            jax.ShapeDtypeStruct((NP, D), jnp.float32),
            jax.ShapeDtypeStruct((NP, D), jnp.float32),
        ],
    )(nt2, wembT, bemb, wT)


def _tc_mid_body(h_ref, p0_ref, p1_ref, b_ref, wt_ref, hn_ref, g_ref):
    h = h_ref[...] + jnp.maximum(p0_ref[...] + p1_ref[...] + b_ref[...], 0.0)
    hn_ref[...] = h
    g_ref[...] = lax.dot_general(h, wt_ref[...], (((1,), (0,)), ((), ())),
                                 precision=_HI, preferred_element_type=jnp.float32)


def _tc_update_matmul(h, p0, p1, b, wT):
    return pl.pallas_call(
        _tc_mid_body,
        grid=(NP // BR,),
        in_specs=[
            pl.BlockSpec((BR, D), lambda i: (i, 0)),
            pl.BlockSpec((BR, D), lambda i: (i, 0)),
            pl.BlockSpec((BR, D), lambda i: (i, 0)),
            pl.BlockSpec((1, D), lambda i: (0, 0)),
            pl.BlockSpec((D, D), lambda i: (0, 0)),
        ],
        out_specs=[
            pl.BlockSpec((BR, D), lambda i: (i, 0)),
            pl.BlockSpec((BR, D), lambda i: (i, 0)),
        ],
        out_shape=[
            jax.ShapeDtypeStruct((NP, D), jnp.float32),
            jax.ShapeDtypeStruct((NP, D), jnp.float32),
        ],
    )(h, p0, p1, b, wT)


def _tc3_body(ag_ref, h_ref, p0_ref, p1_ref, b_ref, s_ref):
    ag = ag_ref[0, 0]
    b = b_ref[...]
    h3 = h_ref[...] + jnp.maximum(p0_ref[...] + p1_ref[...] + b, 0.0)
    agrow = h_ref[pl.ds(ag, 1), :] + jnp.maximum(
        p0_ref[pl.ds(ag, 1), :] + p1_ref[pl.ds(ag, 1), :] + b, 0.0)
    s_ref[...] = lax.dot_general(h3, agrow, (((1,), (1,)), ((), ())),
                                 precision=_HI, preferred_element_type=jnp.float32)


def _tc_final_update_score(ag2, h, p0, p1, b):
    return pl.pallas_call(
        _tc3_body,
        in_specs=[
            pl.BlockSpec(memory_space=pltpu.SMEM),
            pl.BlockSpec((NP, D), lambda: (0, 0)),
            pl.BlockSpec((NP, D), lambda: (0, 0)),
            pl.BlockSpec((NP, D), lambda: (0, 0)),
            pl.BlockSpec((1, D), lambda: (0, 0)),
        ],
        out_specs=pl.BlockSpec((NP, 1), lambda: (0, 0)),
        out_shape=jax.ShapeDtypeStruct((NP, 1), jnp.float32),
    )(ag2, h, p0, p1, b)


def _tc4_body(sc_ref, mask_ref, gum_ref, pi_ref, act_ref):
    inv = jnp.float32(1.0) / jnp.sqrt(jnp.float32(D))
    score = jnp.where(mask_ref[...] != 0, -jnp.inf, sc_ref[...] * inv)
    m = jnp.max(score)
    e = jnp.exp(score - m)
    pi_ref[...] = e / jnp.sum(e)
    v = score + gum_ref[...]
    m2 = jnp.max(v)
    r = lax.broadcasted_iota(jnp.int32, (TP // D, D), 0)
    c = lax.broadcasted_iota(jnp.int32, (TP // D, D), 1)
    flat = r * D + c
    act_ref[0, 0] = jnp.min(jnp.where(v == m2, flat, jnp.int32(2**30)))


def _tc_softmax_sample(sc2, mask2, gum2):
    return pl.pallas_call(
        _tc4_body,
        in_specs=[
            pl.BlockSpec((TP // D, D), lambda: (0, 0)),
            pl.BlockSpec((TP // D, D), lambda: (0, 0)),
            pl.BlockSpec((TP // D, D), lambda: (0, 0)),
        ],
        out_specs=[
            pl.BlockSpec((TP // D, D), lambda: (0, 0)),
            pl.BlockSpec(memory_space=pltpu.SMEM),
        ],
        out_shape=[
            jax.ShapeDtypeStruct((TP // D, D), jnp.float32),
            jax.ShapeDtypeStruct((1, 1), jnp.int32),
        ],
    )(sc2, mask2, gum2)


_MESH = plsc.VectorSubcoreMesh(core_axis_name="c", subcore_axis_name="s")


@functools.partial(
    pl.kernel,
    out_type=jax.ShapeDtypeStruct((NC, NP, D), jnp.float32),
    mesh=_MESH,
    scratch_types=[
        pltpu.VMEM((SB, K), jnp.int32),
        pltpu.VMEM((SB, K), jnp.int32),
        pltpu.VMEM((K, D), jnp.float32),
        pltpu.VMEM((K, D), jnp.float32),
        pltpu.VMEM_SHARED((NP, D), jnp.float32),
    ],
)
def _sc_segsum(g_hbm, src_hbm, dst_hbm, out_hbm, sidx, didx, rows, rows1, acc):
    cid = lax.axis_index("c")
    sid = lax.axis_index("s")
    wid = cid * NS + sid

    @pl.loop(0, K)
    def _(i):
        @pl.loop(0, D // 16)
        def _(j):
            rows.at[i][pl.ds(j * 16, 16)] = jnp.zeros((16,), jnp.float32)

    @pl.loop(0, ROWS_PER_SUB // K)
    def _(i):
        pltpu.sync_copy(rows, acc.at[pl.ds(sid * ROWS_PER_SUB + i * K, K)])

    plsc.subcore_barrier()

    @pl.loop(0, NSB)
    def _(sb):
        pltpu.sync_copy(src_hbm.at[wid].at[pl.ds(sb * SB, SB)], sidx)
        pltpu.sync_copy(dst_hbm.at[wid].at[pl.ds(sb * SB, SB)], didx)

        @pl.loop(0, SB, step=2)
        def _(c):
            def inner(g0, g1, s0, s1):
                d0 = pltpu.make_async_copy(g_hbm.at[sidx.at[c]], rows, g0)
                d1 = pltpu.make_async_copy(g_hbm.at[sidx.at[c + 1]], rows1, g1)
                d0.start()
                d1.start()
                d0.wait()
                w0 = pltpu.make_async_copy(rows, acc.at[didx.at[c]], s0)
                w0.start(add=True)
                d1.wait()
                w1 = pltpu.make_async_copy(rows1, acc.at[didx.at[c + 1]], s1)
                w1.start(add=True)
                w0.wait()
                w1.wait()
            pl.run_scoped(inner,
                          pltpu.SemaphoreType.DMA(()), pltpu.SemaphoreType.DMA(()),
                          pltpu.SemaphoreType.DMA(()), pltpu.SemaphoreType.DMA(()))

    plsc.subcore_barrier()

    pltpu.sync_copy(acc.at[pl.ds(sid * ROWS_PER_SUB, ROWS_PER_SUB)],
                    out_hbm.at[cid].at[pl.ds(sid * ROWS_PER_SUB, ROWS_PER_SUB)])


TIPW = TP // NW

_CP = pltpu.CompilerParams()
if "needs_layout_passes" in pltpu.CompilerParams.__dataclass_fields__:
    _CP = dataclasses.replace(_CP, needs_layout_passes=False)


@functools.partial(
    pl.kernel,
    out_type=jax.ShapeDtypeStruct((TP,), jnp.float32),
    mesh=_MESH,
    compiler_params=_CP,
    scratch_types=[
        pltpu.VMEM((NP,), jnp.float32),
        pltpu.VMEM((TIPW,), jnp.int32),
        pltpu.VMEM((TIPW,), jnp.float32),
    ],
)
def _sc_score_gather(s_hbm, ti_hbm, out_hbm, sv, tiv, ov):
    cid = lax.axis_index("c")
    sid = lax.axis_index("s")
    wid = cid * NS + sid
    pltpu.sync_copy(s_hbm, sv)
    pltpu.sync_copy(ti_hbm.at[wid], tiv)

    @pl.loop(0, TIPW // 16)
    def _(j):
        iv = tiv[pl.ds(j * 16, 16)]
        ov[pl.ds(j * 16, 16)] = plsc.load_gather(sv, [iv])

    pltpu.sync_copy(ov, out_hbm.at[pl.ds(wid * TIPW, TIPW)])


def kernel(node_type, edge_index, ag_node_idx, task_node_indices, finished_task,
           W_emb, b_emb, W_gnn, b_gnn):
    f32, i32 = jnp.float32, jnp.int32
    nt2 = jnp.pad(node_type.astype(i32), (0, NP - N)).reshape(NP, 1)
    pad_e = CH * K - EPW
    if pad_e:
        pad_idx = (jnp.arange(NW * pad_e, dtype=i32) % (NP - N) + N).reshape(NW, pad_e)
        src3 = jnp.concatenate([edge_index[0].astype(i32).reshape(NW, EPW),
                                pad_idx], axis=1).reshape(NW, CH, K)
        dst3 = jnp.concatenate([edge_index[1].astype(i32).reshape(NW, EPW),
                                pad_idx], axis=1).reshape(NW, CH, K)
    else:
        src3 = edge_index[0].astype(i32).reshape(NW, CH, K)
        dst3 = edge_index[1].astype(i32).reshape(NW, CH, K)
    wembT = W_emb.T.astype(f32)
    bemb = b_emb.reshape(1, D).astype(f32)

    h, g = _tc_embed_matmul(nt2, wembT, bemb, W_gnn[0].T)
    for l in range(W_gnn.shape[0]):
        p = _sc_segsum(g, src3, dst3)
        b_l = b_gnn[l].reshape(1, D)
        if l + 1 < W_gnn.shape[0]:
            h, g = _tc_update_matmul(h, p[0], p[1], b_l, W_gnn[l + 1].T)
        else:
            ag2 = jnp.asarray(ag_node_idx, i32).reshape(1, 1)
            s = _tc_final_update_score(ag2, h, p[0], p[1], b_l)

    ti2 = jnp.pad(task_node_indices.astype(i32), (0, TP - T)).reshape(NW, TIPW)
    sc = _sc_score_gather(s.reshape(NP), ti2)

    mask2 = jnp.pad(finished_task, (0, TP - T), constant_values=True)
    mask2 = mask2.astype(i32).reshape(TP // D, D)
    gum2 = jnp.asarray(_GUM2D)
    pi2, act = _tc_softmax_sample(sc.reshape(TP // D, D), mask2, gum2)

    pi = pi2.reshape(TP)[:T, None]
    return act[0, 0], pi
